# Initial kernel scaffold; baseline (speedup 1.0000x reference)
import jax
import jax.numpy as jnp
from jax.experimental import pallas as pl


def kernel(x, conn_rows, conn_cols, spline_w, base_w):
    raise NotImplementedError("write your pallas kernel here")


# speedup vs baseline: 6.4496x; 6.4496x over previous
import functools

import jax
import jax.numpy as jnp
from jax import lax
from jax.experimental import pallas as pl
from jax.experimental.pallas import tpu as pltpu
from jax.experimental.pallas import tpu_sc as plsc

B = 64
F = 16384
O = 16384
NNZ = 268435
G = 8
Q = F // G
GRID_MIN = -2.0
GRID_MAX = 2.0
DENOM = (GRID_MAX - GRID_MIN) / (G - 1)
INV_DENOM = 1.0 / DENOM
GRID_STEP = (GRID_MAX - GRID_MIN) / (G - 1)

NC = 2
NS = 16
NT = NC * NS
E_T = 8448
EPAD = NT * E_T
C = 16
NCHUNK = E_T // C
PAIRS = NCHUNK // 2


def _sc_body(xg_h, xt_h, cols_h, rows_h, sw_h, bw_h, zero_h, part_h,
             acc_s, cols_v, rows_v, bw_v,
             xb0, xb1, xt0, xt1, sw0, sw1, res,
             sxg0, sxg1, sxt0, sxt1, ssw0, ssw1):
    cid = lax.axis_index("c")
    sid = lax.axis_index("s")
    tid = cid * NS + sid
    ebase = tid * E_T

    pltpu.sync_copy(cols_h.at[pl.ds(ebase, E_T)], cols_v)
    pltpu.sync_copy(rows_h.at[pl.ds(ebase, E_T)], rows_v)
    pltpu.sync_copy(bw_h.at[pl.ds(ebase, E_T)], bw_v)

    rows_per_tile = O // NS
    rbase = sid * rows_per_tile
    pltpu.sync_copy(zero_h.at[pl.ds(rbase, rows_per_tile)],
                    acc_s.at[pl.ds(rbase, rows_per_tile)])
    plsc.subcore_barrier()

    def fire(ci, xb, xt_b, swb, s_xg, s_xt, s_sw):
        off = ci * C
        cvec = cols_v[pl.ds(off, C)]
        qvec = lax.shift_right_logical(cvec, 3)
        pltpu.async_copy(xg_h.at[qvec], xb, s_xg)
        pltpu.async_copy(xt_h.at[cvec], xt_b, s_xt)
        pltpu.async_copy(sw_h.at[pl.ds((ebase + off) * G, C * G)], swb, s_sw)

    def compute(ci, xb, xt_b, swb, s_xg, s_xt, s_sw):
        off = ci * C
        cvec = cols_v[pl.ds(off, C)]
        kvec = jnp.bitwise_and(cvec, 7)
        grvec = kvec.astype(jnp.float32) * GRID_STEP + GRID_MIN
        bwvec = bw_v[pl.ds(off, C)]
        pltpu.make_async_copy(xg_h.at[pl.ds(0, C)], xb, s_xg).wait()
        pltpu.make_async_copy(xt_h.at[pl.ds(0, C)], xt_b, s_xt).wait()
        pltpu.make_async_copy(sw_h.at[pl.ds(0, C * G)], swb, s_sw).wait()

        for p in range(C // 2):
            swv = swb[pl.ds(2 * p * G, 2 * G)]
            for i, wb in ((2 * p, 0), (2 * p + 1, G)):
                gr = grvec[i]
                bw = bwvec[i]
                for v in range(B // 16):
                    sx = xt_b[i, pl.ds(v * 16, 16)]
                    accv = bw * (sx / (1.0 + jnp.exp(-sx)))
                    for g in range(G):
                        xv = xb[i, g, pl.ds(v * 16, 16)]
                        t = (xv - gr) * INV_DENOM
                        accv = accv + swv[wb + g] * jnp.exp(-(t * t))
                    res[i, pl.ds(v * 16, 16)] = accv

        rvec = rows_v[pl.ds(off, C)]
        pltpu.sync_copy(res, acc_s.at[rvec], add=True)

    bufs0 = (xb0, xt0, sw0, sxg0, sxt0, ssw0)
    bufs1 = (xb1, xt1, sw1, sxg1, sxt1, ssw1)

    fire(0, *bufs0)

    def pair_body(pi, _):
        ci = 2 * pi
        fire(ci + 1, *bufs1)
        compute(ci, *bufs0)
        fire(jnp.minimum(ci + 2, NCHUNK - 1), *bufs0)
        compute(ci + 1, *bufs1)
        return 0

    lax.fori_loop(0, PAIRS, pair_body, 0)

    pltpu.make_async_copy(xg_h.at[pl.ds(0, C)], xb0, sxg0).wait()
    pltpu.make_async_copy(xt_h.at[pl.ds(0, C)], xt0, sxt0).wait()
    pltpu.make_async_copy(sw_h.at[pl.ds(0, C * G)], sw0, ssw0).wait()

    plsc.subcore_barrier()
    pltpu.sync_copy(acc_s.at[pl.ds(rbase, rows_per_tile)],
                    part_h.at[cid, pl.ds(rbase, rows_per_tile)])


def _combine_body(part_ref, out_ref):
    s = part_ref[0] + part_ref[1]
    out_ref[...] = s.T


def kernel(x, conn_rows, conn_cols, spline_w, base_w):
    xg = x.reshape(B, G, Q).transpose(2, 1, 0)
    xt = x.T
    pad = EPAD - NNZ
    cols_p = jnp.concatenate([conn_cols, jnp.zeros((pad,), jnp.int32)])
    rows_p = jnp.concatenate([conn_rows, jnp.zeros((pad,), jnp.int32)])
    sw_p = jnp.concatenate([spline_w, jnp.zeros((pad, G), jnp.float32)]).reshape(EPAD * G)
    bw_p = jnp.concatenate([base_w, jnp.zeros((pad,), jnp.float32)])
    zero = jnp.zeros((O, B), jnp.float32)

    mesh = plsc.VectorSubcoreMesh(core_axis_name="c", subcore_axis_name="s")
    sc_call = pl.kernel(
        _sc_body,
        out_type=jax.ShapeDtypeStruct((NC, O, B), jnp.float32),
        mesh=mesh,
        scratch_types=[
            pltpu.VMEM_SHARED((O, B), jnp.float32),
            pltpu.VMEM((E_T,), jnp.int32),
            pltpu.VMEM((E_T,), jnp.int32),
            pltpu.VMEM((E_T,), jnp.float32),
            pltpu.VMEM((C, G, B), jnp.float32),
            pltpu.VMEM((C, G, B), jnp.float32),
            pltpu.VMEM((C, B), jnp.float32),
            pltpu.VMEM((C, B), jnp.float32),
            pltpu.VMEM((C * G,), jnp.float32),
            pltpu.VMEM((C * G,), jnp.float32),
            pltpu.VMEM((C, B), jnp.float32),
            pltpu.SemaphoreType.DMA,
            pltpu.SemaphoreType.DMA,
            pltpu.SemaphoreType.DMA,
            pltpu.SemaphoreType.DMA,
            pltpu.SemaphoreType.DMA,
            pltpu.SemaphoreType.DMA,
        ],
        compiler_params=pltpu.CompilerParams(use_tc_tiling_on_sc=False),
    )
    part = sc_call(xg, xt, cols_p, rows_p, sw_p, bw_p, zero)

    BLK = 512
    out = pl.pallas_call(
        _combine_body,
        grid=(O // BLK,),
        in_specs=[pl.BlockSpec((NC, BLK, B), lambda i: (0, i, 0))],
        out_specs=pl.BlockSpec((B, BLK), lambda i: (0, i)),
        out_shape=jax.ShapeDtypeStruct((B, O), jnp.float32),
    )(part)
    return out


# speedup vs baseline: 12.1068x; 1.8771x over previous
import functools

import jax
import jax.numpy as jnp
from jax import lax
from jax.experimental import pallas as pl
from jax.experimental.pallas import tpu as pltpu
from jax.experimental.pallas import tpu_sc as plsc

B = 64
F = 16384
O = 16384
NNZ = 268435
G = 8
Q = F // G
GRID_MIN = -2.0
GRID_MAX = 2.0
GRID_STEP = (GRID_MAX - GRID_MIN) / (G - 1)
INV_DENOM = 1.0 / GRID_STEP
TW = G * B + B

NC = 2
NS = 16
NT = NC * NS
E_T = 8448
EPAD = NT * E_T
C = 16
NCHUNK = E_T // C
PAIRS = NCHUNK // 2

QB = 256


def _table_body(xg_ref, xkq_ref, tab_ref):
    k = pl.program_id(0)
    gr = GRID_MIN + k.astype(jnp.float32) * GRID_STEP
    t = (xg_ref[...] - gr) * INV_DENOM
    tab_ref[:, : G * B] = jnp.exp(-(t * t))
    sx = xkq_ref[...]
    tab_ref[:, G * B :] = sx * jax.nn.sigmoid(sx)


def _sc_body(tab_h, cols_h, rows_h, sw_h, bw_h, zero_h, part_h,
             acc_s, cols_v, rows_v, bw_v,
             xb0, xb1, sw0, sw1, res,
             sxg0, sxg1, ssw0, ssw1):
    cid = lax.axis_index("c")
    sid = lax.axis_index("s")
    tid = cid * NS + sid
    ebase = tid * E_T

    pltpu.sync_copy(cols_h.at[pl.ds(ebase, E_T)], cols_v)
    pltpu.sync_copy(rows_h.at[pl.ds(ebase, E_T)], rows_v)
    pltpu.sync_copy(bw_h.at[pl.ds(ebase, E_T)], bw_v)

    rows_per_tile = O // NS
    rbase = sid * rows_per_tile
    pltpu.sync_copy(zero_h.at[pl.ds(rbase, rows_per_tile)],
                    acc_s.at[pl.ds(rbase, rows_per_tile)])
    plsc.subcore_barrier()

    def fire(ci, xb, swb, s_xg, s_sw):
        off = ci * C
        cvec = cols_v[pl.ds(off, C)]
        jvec = jnp.bitwise_and(cvec, 7) * Q + lax.shift_right_logical(cvec, 3)
        pltpu.async_copy(tab_h.at[jvec], xb, s_xg)
        pltpu.async_copy(sw_h.at[pl.ds((ebase + off) * G, C * G)], swb, s_sw)

    def compute(ci, xb, swb, s_xg, s_sw):
        off = ci * C
        bwvec = bw_v[pl.ds(off, C)]
        pltpu.make_async_copy(tab_h.at[pl.ds(0, C)], xb, s_xg).wait()
        pltpu.make_async_copy(sw_h.at[pl.ds(0, C * G)], swb, s_sw).wait()

        for p in range(C // 2):
            swv = swb[pl.ds(2 * p * G, 2 * G)]
            for i, wb in ((2 * p, 0), (2 * p + 1, G)):
                bw = bwvec[i]
                for v in range(B // 16):
                    accv = bw * xb[i, pl.ds(G * B + v * 16, 16)]
                    for g in range(G):
                        accv = accv + swv[wb + g] * xb[i, pl.ds(g * B + v * 16, 16)]
                    res[i, pl.ds(v * 16, 16)] = accv

        rvec = rows_v[pl.ds(off, C)]
        pltpu.sync_copy(res, acc_s.at[rvec], add=True)

    bufs0 = (xb0, sw0, sxg0, ssw0)
    bufs1 = (xb1, sw1, sxg1, ssw1)

    fire(0, *bufs0)

    def pair_body(pi, _):
        ci = 2 * pi
        fire(ci + 1, *bufs1)
        compute(ci, *bufs0)
        fire(jnp.minimum(ci + 2, NCHUNK - 1), *bufs0)
        compute(ci + 1, *bufs1)
        return 0

    lax.fori_loop(0, PAIRS, pair_body, 0)

    pltpu.make_async_copy(tab_h.at[pl.ds(0, C)], xb0, sxg0).wait()
    pltpu.make_async_copy(sw_h.at[pl.ds(0, C * G)], sw0, ssw0).wait()

    plsc.subcore_barrier()
    pltpu.sync_copy(acc_s.at[pl.ds(rbase, rows_per_tile)],
                    part_h.at[cid, pl.ds(rbase, rows_per_tile)])


def _combine_body(part_ref, out_ref):
    s = part_ref[0] + part_ref[1]
    out_ref[...] = s.T


def kernel(x, conn_rows, conn_cols, spline_w, base_w):
    xg = x.reshape(B, G, Q).transpose(2, 1, 0).reshape(Q, G * B)
    xkq = x.T.reshape(Q, G, B).transpose(1, 0, 2).reshape(F, B)
    pad = EPAD - NNZ
    cols_p = jnp.concatenate([conn_cols, jnp.zeros((pad,), jnp.int32)])
    rows_p = jnp.concatenate([conn_rows, jnp.zeros((pad,), jnp.int32)])
    sw_p = jnp.concatenate([spline_w, jnp.zeros((pad, G), jnp.float32)]).reshape(EPAD * G)
    bw_p = jnp.concatenate([base_w, jnp.zeros((pad,), jnp.float32)])
    zero = jnp.zeros((O, B), jnp.float32)

    tab = pl.pallas_call(
        _table_body,
        grid=(G, Q // QB),
        in_specs=[
            pl.BlockSpec((QB, G * B), lambda k, i: (i, 0)),
            pl.BlockSpec((QB, B), lambda k, i: (k * (Q // QB) + i, 0)),
        ],
        out_specs=pl.BlockSpec((QB, TW), lambda k, i: (k * (Q // QB) + i, 0)),
        out_shape=jax.ShapeDtypeStruct((F, TW), jnp.float32),
    )(xg, xkq)

    mesh = plsc.VectorSubcoreMesh(core_axis_name="c", subcore_axis_name="s")
    sc_call = pl.kernel(
        _sc_body,
        out_type=jax.ShapeDtypeStruct((NC, O, B), jnp.float32),
        mesh=mesh,
        scratch_types=[
            pltpu.VMEM_SHARED((O, B), jnp.float32),
            pltpu.VMEM((E_T,), jnp.int32),
            pltpu.VMEM((E_T,), jnp.int32),
            pltpu.VMEM((E_T,), jnp.float32),
            pltpu.VMEM((C, TW), jnp.float32),
            pltpu.VMEM((C, TW), jnp.float32),
            pltpu.VMEM((C * G,), jnp.float32),
            pltpu.VMEM((C * G,), jnp.float32),
            pltpu.VMEM((C, B), jnp.float32),
            pltpu.SemaphoreType.DMA,
            pltpu.SemaphoreType.DMA,
            pltpu.SemaphoreType.DMA,
            pltpu.SemaphoreType.DMA,
        ],
        compiler_params=pltpu.CompilerParams(use_tc_tiling_on_sc=False),
    )
    part = sc_call(tab, cols_p, rows_p, sw_p, bw_p, zero)

    BLK = 512
    out = pl.pallas_call(
        _combine_body,
        grid=(O // BLK,),
        in_specs=[pl.BlockSpec((NC, BLK, B), lambda i: (0, i, 0))],
        out_specs=pl.BlockSpec((B, BLK), lambda i: (0, i)),
        out_shape=jax.ShapeDtypeStruct((B, O), jnp.float32),
    )(part)
    return out


# speedup vs baseline: 12.8226x; 1.0591x over previous
import jax
import jax.numpy as jnp
from jax import lax
from jax.experimental import pallas as pl
from jax.experimental.pallas import tpu as pltpu
from jax.experimental.pallas import tpu_sc as plsc

B = 64
F = 16384
O = 16384
NNZ = 268435
G = 8
Q = F // G
GRID_MIN = -2.0
GRID_MAX = 2.0
GRID_STEP = (GRID_MAX - GRID_MIN) / (G - 1)
INV_DENOM = 1.0 / GRID_STEP
TW = G * B + B
AW = 16

NC = 2
NS = 16
NT = NC * NS
E_T = 8448
EPAD = NT * E_T
C = 16
NCHUNK = E_T // C
DEPTH = 2

QB = 256


def _table_body(xg_ref, xkq_ref, tab_ref):
    k = pl.program_id(0)
    gr = GRID_MIN + k.astype(jnp.float32) * GRID_STEP
    t = (xg_ref[...] - gr) * INV_DENOM
    tab_ref[:, : G * B] = jnp.exp(-(t * t))
    sx = xkq_ref[...]
    tab_ref[:, G * B :] = sx * jax.nn.sigmoid(sx)


def _sc_body(tab_h, cols_h, rows_h, aux_h, part_h,
             acc_s, cols_v, rows_v, xbs0, xbs1, axs0, axs1,
             res0, res1, sg0, sg1, ssc):
    cid = lax.axis_index("c")
    sid = lax.axis_index("s")
    tid = cid * NS + sid
    ebase = tid * E_T
    xbs = (xbs0, xbs1)
    axs = (axs0, axs1)
    sgs = (sg0, sg1)
    ress = (res0, res1)

    pltpu.sync_copy(cols_h.at[pl.ds(ebase, E_T)], cols_v)
    pltpu.sync_copy(rows_h.at[pl.ds(ebase, E_T)], rows_v)

    zv = jnp.zeros((16,), jnp.float32)
    for i in range(C):
        for v in range(B // 16):
            res0[i, pl.ds(v * 16, 16)] = zv
    rows_per_tile = O // NS
    rbase = sid * rows_per_tile

    def zbody(t, _):
        pltpu.sync_copy(res0, acc_s.at[pl.ds(rbase + t * C, C)])
        return 0

    lax.fori_loop(0, rows_per_tile // C, zbody, 0)
    plsc.subcore_barrier()

    def fire(ci, p):
        off = ci * C
        cvec = cols_v[pl.ds(off, C)]
        jvec = jnp.bitwise_and(cvec, 7) * Q + lax.shift_right_logical(cvec, 3)
        pltpu.async_copy(tab_h.at[jvec], xbs[p], sgs[p])
        pltpu.async_copy(aux_h.at[pl.ds((ebase + off) * AW, C * AW)], axs[p], sgs[p])

    def compute(ci, p, rp, first):
        xb = xbs[p]
        ax = axs[p]
        res = ress[rp]
        pltpu.make_async_copy(tab_h.at[pl.ds(0, C)], xb, sgs[p]).wait()
        pltpu.make_async_copy(aux_h.at[pl.ds(0, C * AW)], ax, sgs[p]).wait()

        for i in range(C):
            rec = ax[pl.ds(i * AW, 16)]
            bw = rec[8]
            for v in range(B // 16):
                accv = bw * xb[i, pl.ds(G * B + v * 16, 16)]
                for g in range(G):
                    accv = accv + rec[g] * xb[i, pl.ds(g * B + v * 16, 16)]
                res[i, pl.ds(v * 16, 16)] = accv

        rvec = rows_v[pl.ds(ci * C, C)]
        if first is None:
            pltpu.make_async_copy(part_h.at[0, pl.ds(0, C)], res, ssc).wait()
        else:
            @pl.when(jnp.logical_not(first))
            def _():
                pltpu.make_async_copy(part_h.at[0, pl.ds(0, C)], res, ssc).wait()
        pltpu.async_copy(res, acc_s.at[rvec], ssc, add=True)

    for ci in range(DEPTH - 1):
        fire(ci, ci)

    def quad_body(pi, _):
        ci0 = pi * DEPTH
        for s in range(DEPTH):
            ci = ci0 + s
            fire(jnp.minimum(ci + DEPTH - 1, NCHUNK - 1), (s + DEPTH - 1) % DEPTH)
            compute(ci, s, s % 2, (pi == 0) if s == 0 else None)
        return 0

    lax.fori_loop(0, NCHUNK // DEPTH, quad_body, 0)

    pltpu.make_async_copy(part_h.at[0, pl.ds(0, C)], res0, ssc).wait()
    for p in range(DEPTH - 1):
        pltpu.make_async_copy(tab_h.at[pl.ds(0, C)], xbs[p], sgs[p]).wait()
        pltpu.make_async_copy(aux_h.at[pl.ds(0, C * AW)], axs[p], sgs[p]).wait()

    plsc.subcore_barrier()
    pltpu.sync_copy(acc_s.at[pl.ds(rbase, rows_per_tile)],
                    part_h.at[cid, pl.ds(rbase, rows_per_tile)])


def _combine_body(part_ref, out_ref):
    s = part_ref[0] + part_ref[1]
    out_ref[...] = s.T


def kernel(x, conn_rows, conn_cols, spline_w, base_w):
    xg = x.reshape(B, G, Q).transpose(2, 1, 0).reshape(Q, G * B)
    xkq = x.T.reshape(Q, G, B).transpose(1, 0, 2).reshape(F, B)
    pad = EPAD - NNZ
    rows_p = jnp.concatenate([conn_rows, jnp.zeros((pad,), jnp.int32)])
    cols_p = jnp.concatenate([conn_cols, jnp.zeros((pad,), jnp.int32)])
    sw_p = jnp.concatenate([spline_w, jnp.zeros((pad, G), jnp.float32)])
    bw_p = jnp.concatenate([base_w, jnp.zeros((pad,), jnp.float32)])
    aux = jnp.concatenate(
        [sw_p, bw_p[:, None], lax.bitcast_convert_type(rows_p, jnp.float32)[:, None],
         jnp.zeros((EPAD, AW - G - 2), jnp.float32)], axis=1).reshape(EPAD * AW)

    tab = pl.pallas_call(
        _table_body,
        grid=(G, Q // QB),
        in_specs=[
            pl.BlockSpec((QB, G * B), lambda k, i: (i, 0)),
            pl.BlockSpec((QB, B), lambda k, i: (k * (Q // QB) + i, 0)),
        ],
        out_specs=pl.BlockSpec((QB, TW), lambda k, i: (k * (Q // QB) + i, 0)),
        out_shape=jax.ShapeDtypeStruct((F, TW), jnp.float32),
    )(xg, xkq)

    mesh = plsc.VectorSubcoreMesh(core_axis_name="c", subcore_axis_name="s")
    sc_call = pl.kernel(
        _sc_body,
        out_type=jax.ShapeDtypeStruct((NC, O, B), jnp.float32),
        mesh=mesh,
        scratch_types=(
            [pltpu.VMEM_SHARED((O, B), jnp.float32),
             pltpu.VMEM((E_T,), jnp.int32),
             pltpu.VMEM((E_T,), jnp.int32)]
            + [pltpu.VMEM((C, TW), jnp.float32) for _ in range(DEPTH)]
            + [pltpu.VMEM((C * AW,), jnp.float32) for _ in range(DEPTH)]
            + [pltpu.VMEM((C, B), jnp.float32) for _ in range(2)]
            + [pltpu.SemaphoreType.DMA for _ in range(DEPTH + 1)]
        ),
        compiler_params=pltpu.CompilerParams(use_tc_tiling_on_sc=False),
    )
    part = sc_call(tab, cols_p, rows_p, aux)

    BLK = 512
    out = pl.pallas_call(
        _combine_body,
        grid=(O // BLK,),
        in_specs=[pl.BlockSpec((NC, BLK, B), lambda i: (0, i, 0))],
        out_specs=pl.BlockSpec((B, BLK), lambda i: (0, i)),
        out_shape=jax.ShapeDtypeStruct((B, O), jnp.float32),
    )(part)
    return out
